# baseline (speedup 1.0000x reference)
import jax
import jax.numpy as jnp
from jax import lax
from jax.experimental import pallas as pl
from jax.experimental.pallas import tpu as pltpu
from jax.experimental.pallas import tpu_sc as plsc

N_FIELDS = 26
VOCAB = 100000
EMBED_DIM = 32
BATCH = 16384

NC, NS = 2, 16
NW = NC * NS
LANES = 16
ICHUNK = 2048
NI = BATCH // ICHUNK
NIDX = N_FIELDS * BATCH
RFIELDS = 9
ROUNDS = ((0, 9), (9, 9), (18, 8))
HALF = VOCAB // 2
NPLANE = N_FIELDS * EMBED_DIM


def _emb_body(tab_hbm, xt_hbm, out_hbm, plane0_v, plane1_v, idx_v, acc_v,
              stage_sp, sem_p, sem_i):
    sid = lax.axis_index("s")
    dw = sid * NC + lax.axis_index("c")
    last_word = (NPLANE - 1) * VOCAB

    planes = (plane0_v, plane1_v)

    def fill(f, h, buf):
        off = jnp.minimum((f * EMBED_DIM + dw) * VOCAB, last_word) + h * HALF
        return pltpu.async_copy(tab_hbm.at[pl.ds(off, HALF)],
                                planes[buf], sem_p)

    def wait_fill():
        pltpu.make_async_copy(tab_hbm.at[pl.ds(0, HALF)], plane0_v,
                              sem_p).wait()

    def do_half(base_f, fr, h, first):
        wait_fill()
        f_next = base_f + fr + h
        fill(f_next, 1 - h, 1 - h)

        descs = [pltpu.async_copy(stage_sp.at[pl.ds(fr * BATCH, ICHUNK)],
                                  idx_v.at[0], sem_i)]
        for c in range(NI):
            if c + 1 < NI:
                descs.append(pltpu.async_copy(
                    stage_sp.at[pl.ds(fr * BATCH + (c + 1) * ICHUNK, ICHUNK)],
                    idx_v.at[(c + 1) % 2], sem_i))
            descs[c].wait()

            @plsc.parallel_loop(0, ICHUNK // LANES, unroll=8)
            def gather_body(i):
                s = c * ICHUNK + i * LANES
                v = idx_v[c % 2, pl.ds(i * LANES, LANES)]
                if h == 0:
                    m = v < HALF
                    local = jnp.minimum(v, HALF - 1)
                else:
                    m = v >= HALF
                    local = jnp.maximum(v - HALF, 0)
                g = plsc.load_gather(planes[h], [local], mask=m)
                g = jnp.where(m, g, 0.0)
                if first:
                    acc_v[pl.ds(s, LANES)] = g
                else:
                    plsc.addupdate(acc_v.at[pl.ds(s, LANES)], g)

    def do_field(base_f, fr, first):
        do_half(base_f, fr, 0, first)
        do_half(base_f, fr, 1, False)

    fill(0, 0, 0)
    for base_f, nf in ROUNDS:
        if base_f > 0:
            plsc.subcore_barrier()
        share = nf * BATCH // NS
        pltpu.sync_copy(
            xt_hbm.at[pl.ds(base_f * BATCH + sid * share, share)],
            stage_sp.at[pl.ds(sid * share, share)])
        plsc.subcore_barrier()

        if base_f == 0:
            do_field(0, 0, True)

            def field_body(fr, carry):
                do_field(0, fr, False)
                return carry

            lax.fori_loop(1, nf, field_body, 0)
        else:
            def field_body(fr, carry):
                do_field(base_f, fr, False)
                return carry

            lax.fori_loop(0, nf, field_body, 0)

    wait_fill()
    pltpu.sync_copy(acc_v, out_hbm.at[dw])


def kernel(x_categorical, tables):
    tab = jnp.transpose(tables, (0, 2, 1)).reshape(NPLANE * VOCAB)
    xt = jnp.transpose(x_categorical).reshape(NIDX)

    run = pl.kernel(
        _emb_body,
        out_type=jax.ShapeDtypeStruct((EMBED_DIM, BATCH), jnp.float32),
        mesh=plsc.VectorSubcoreMesh(
            core_axis_name="c", subcore_axis_name="s",
            num_cores=NC, num_subcores=NS),
        scratch_types=[
            pltpu.VMEM((HALF,), jnp.float32),
            pltpu.VMEM((HALF,), jnp.float32),
            pltpu.VMEM((2, ICHUNK), jnp.int32),
            pltpu.VMEM((BATCH,), jnp.float32),
            pltpu.VMEM_SHARED((RFIELDS * BATCH,), jnp.int32),
            pltpu.SemaphoreType.DMA,
            pltpu.SemaphoreType.DMA,
        ],
        compiler_params=pltpu.CompilerParams(needs_layout_passes=False),
    )
    return jnp.transpose(run(tab, xt))


# speedup vs baseline: 3.0660x; 3.0660x over previous
import jax
import jax.numpy as jnp
from jax import lax
from jax.experimental import pallas as pl
from jax.experimental.pallas import tpu as pltpu
from jax.experimental.pallas import tpu_sc as plsc

N_FIELDS = 26
VOCAB = 100000
EMBED_DIM = 32
BATCH = 16384

NC, NS = 2, 16
NW = NC * NS
LANES = 16
ICHUNK = 2048
NI = BATCH // ICHUNK
NIDX = N_FIELDS * BATCH
RFIELDS = 9
ROUNDS = ((0, 9), (9, 9), (18, 8))


def _emb_body(tab_hbm, xt_hbm, out_hbm, plane_v, idx_v, acc_v, stage_sp,
              sem_i):
    sid = lax.axis_index("s")
    dw = sid * NC + lax.axis_index("c")

    def do_field(base_f, fr, first):
        pltpu.sync_copy(tab_hbm.at[(base_f * EMBED_DIM + fr * EMBED_DIM) + dw],
                        plane_v)
        descs = [pltpu.async_copy(stage_sp.at[pl.ds(fr * BATCH, ICHUNK)],
                                  idx_v.at[0], sem_i)]
        for c in range(NI):
            if c + 1 < NI:
                descs.append(pltpu.async_copy(
                    stage_sp.at[pl.ds(fr * BATCH + (c + 1) * ICHUNK, ICHUNK)],
                    idx_v.at[(c + 1) % 2], sem_i))
            descs[c].wait()

            @plsc.parallel_loop(0, ICHUNK // LANES, unroll=8)
            def gather_body(i):
                s = c * ICHUNK + i * LANES
                g = plsc.load_gather(
                    plane_v, [idx_v[c % 2, pl.ds(i * LANES, LANES)]])
                if first:
                    acc_v[pl.ds(s, LANES)] = g
                else:
                    plsc.addupdate(acc_v.at[pl.ds(s, LANES)], g)

    for base_f, nf in ROUNDS:
        if base_f > 0:
            plsc.subcore_barrier()
        share = nf * BATCH // NS
        pltpu.sync_copy(
            xt_hbm.at[pl.ds(base_f * BATCH + sid * share, share)],
            stage_sp.at[pl.ds(sid * share, share)])
        plsc.subcore_barrier()

        if base_f == 0:
            do_field(0, 0, True)

            def field_body(fr, carry):
                do_field(0, fr, False)
                return carry

            lax.fori_loop(1, nf, field_body, 0)
        else:
            def field_body(fr, carry):
                do_field(base_f, fr, False)
                return carry

            lax.fori_loop(0, nf, field_body, 0)

    pltpu.sync_copy(acc_v, out_hbm.at[dw])


def kernel(x_categorical, tables):
    tab = jnp.transpose(tables, (0, 2, 1)).reshape(N_FIELDS * EMBED_DIM, VOCAB)
    xt = jnp.transpose(x_categorical).reshape(NIDX)

    run = pl.kernel(
        _emb_body,
        out_type=jax.ShapeDtypeStruct((EMBED_DIM, BATCH), jnp.float32),
        mesh=plsc.VectorSubcoreMesh(
            core_axis_name="c", subcore_axis_name="s",
            num_cores=NC, num_subcores=NS),
        scratch_types=[
            pltpu.VMEM((VOCAB,), jnp.float32),
            pltpu.VMEM((2, ICHUNK), jnp.int32),
            pltpu.VMEM((BATCH,), jnp.float32),
            pltpu.VMEM_SHARED((RFIELDS * BATCH,), jnp.int32),
            pltpu.SemaphoreType.DMA,
        ],
        compiler_params=pltpu.CompilerParams(needs_layout_passes=False),
    )
    return jnp.transpose(run(tab, xt))
